# baseline (speedup 1.0000x reference)
import functools

import jax
import jax.numpy as jnp
from jax import lax
from jax.experimental import pallas as pl
from jax.experimental.pallas import tpu as pltpu
from jax.experimental.pallas import tpu_sc as plsc

_N = 10000
_DE = 16
_NW = 32
_G = 128
_E = 320000
_NG = _E // _G
_GQ = _NG // _NW
_GR = _NG % _NW
_CG = 16
_NGP = 2504
_NPAD = 10240
_NPS = _NPAD // 16


def _sc_segment_sum(idx2, attr3, zeros):
    mesh = plsc.VectorSubcoreMesh(core_axis_name="c", subcore_axis_name="s")

    @functools.partial(
        pl.kernel,
        mesh=mesh,
        out_type=jax.ShapeDtypeStruct((2, _NPAD, _DE), jnp.float32),
        scratch_types=[
            pltpu.VMEM((_GQ + 2, _G), jnp.int32),
            pltpu.VMEM((_CG, _G, _DE), jnp.float32),
            pltpu.VMEM_SHARED((_NPAD, _DE), jnp.float32),
        ],
        compiler_params=pltpu.CompilerParams(use_tc_tiling_on_sc=False),
    )
    def seg_sum(idx_hbm, attr_hbm, zeros_hbm, out_hbm, idx_v, attr_v, acc):
        c = lax.axis_index("c")
        s = lax.axis_index("s")
        w = s * 2 + c
        ng = _GQ + jnp.where(w < _GR, 1, 0)
        start = _GQ * w + jnp.minimum(w, _GR)
        pltpu.sync_copy(zeros_hbm.at[pl.ds(s * _NPS, _NPS)],
                        acc.at[pl.ds(s * _NPS, _NPS)])
        pltpu.sync_copy(idx_hbm.at[pl.ds(start, _GQ + 2)], idx_v)
        plsc.subcore_barrier()

        def chunk_body(k, _):
            pltpu.sync_copy(attr_hbm.at[pl.ds(start + k * _CG, _CG)], attr_v)

            def slab_body(j, _):
                pltpu.sync_copy(attr_v.at[j],
                                acc.at[idx_v.at[k * _CG + j]],
                                add=True)
                return 0

            lax.fori_loop(0, _CG, slab_body, 0)
            return 0

        lax.fori_loop(0, 4, chunk_body, 0)

        pltpu.sync_copy(attr_hbm.at[pl.ds(start + ng - _CG, _CG)], attr_v)
        off = (_CG + 64) - ng

        def tail_body(j, _):
            pltpu.sync_copy(attr_v.at[off + j],
                            acc.at[idx_v.at[64 + j]],
                            add=True)
            return 0

        lax.fori_loop(0, ng - 64, tail_body, 0)

        plsc.subcore_barrier()
        pltpu.sync_copy(acc.at[pl.ds(s * _NPS, _NPS)],
                        out_hbm.at[c, pl.ds(s * _NPS, _NPS)])

    return seg_sum(idx2, attr3, zeros)


def _tc_mlp_body(nf_ref, p0_ref, p1_ref, w1a_ref, w1b_ref, w2_ref,
                 b1_ref, b2_ref, o_ref):
    agg = p0_ref[...] + p1_ref[...]
    h = jnp.dot(nf_ref[...], w1a_ref[...], preferred_element_type=jnp.float32)
    h = h + jnp.dot(agg, w1b_ref[...], preferred_element_type=jnp.float32)
    h = h + b1_ref[...]
    o = jnp.dot(h, w2_ref[...], preferred_element_type=jnp.float32)
    o_ref[...] = o + b2_ref[...]


def _tc_mlp(node_feats, partials, W1, b1, W2, b2):
    n, d = node_feats.shape
    h_nf = W1.shape[1]
    out_nf = W2.shape[1]
    W1a = W1[:d]
    W1b = W1[d:]
    p0 = partials[0]
    p1 = partials[1]
    blk = 2000
    grid = (n // blk,)
    return pl.pallas_call(
        _tc_mlp_body,
        grid=grid,
        in_specs=[
            pl.BlockSpec((blk, d), lambda i: (i, 0)),
            pl.BlockSpec((blk, _DE), lambda i: (i, 0)),
            pl.BlockSpec((blk, _DE), lambda i: (i, 0)),
            pl.BlockSpec((d, h_nf), lambda i: (0, 0)),
            pl.BlockSpec((_DE, h_nf), lambda i: (0, 0)),
            pl.BlockSpec((h_nf, out_nf), lambda i: (0, 0)),
            pl.BlockSpec((1, h_nf), lambda i: (0, 0)),
            pl.BlockSpec((1, out_nf), lambda i: (0, 0)),
        ],
        out_specs=pl.BlockSpec((blk, out_nf), lambda i: (i, 0)),
        out_shape=jax.ShapeDtypeStruct((n, out_nf), jnp.float32),
    )(node_feats, p0, p1, W1a, W1b, W2,
      b1.reshape(1, h_nf), b2.reshape(1, out_nf))


@jax.jit
def kernel(node_feats, edge_index, edge_attr, W1, b1, W2, b2):
    row = edge_index[0].astype(jnp.int32)
    idx2 = jnp.pad(row, (0, _NGP * _G - _E)).reshape(_NGP, _G)
    attr3 = edge_attr.reshape(_NG, _G, _DE)
    zeros = jnp.zeros((_NPAD, _DE), jnp.float32)
    partials = _sc_segment_sum(idx2, attr3, zeros)[:, :_N]
    return _tc_mlp(node_feats, partials, W1, b1, W2, b2)
